# Initial kernel scaffold; baseline (speedup 1.0000x reference)
import jax
import jax.numpy as jnp
from jax.experimental import pallas as pl


def kernel(QR, symbols, W, b):
    raise NotImplementedError("write your pallas kernel here")


# speedup vs baseline: 2.0762x; 2.0762x over previous
import functools

import jax
import jax.numpy as jnp
from jax import lax
from jax.experimental import pallas as pl
from jax.experimental.pallas import tpu as pltpu
from jax.experimental.pallas import tpu_sc as plsc

_V = 63
_D = 64
_VPAD = 64
_IDX_PER_STREAM = 128


def _table_body(sym_ref, w_ref, b_ref, out_ref):
    out_ref[...] = (
        jnp.dot(sym_ref[...], w_ref[...], preferred_element_type=jnp.float32)
        + b_ref[0:1, :]
    )


def _fused_table(symbols, W, b):
    sym = jnp.pad(symbols, ((0, _VPAD - symbols.shape[0]), (0, 0)))
    b2 = jnp.broadcast_to(b.reshape(1, -1), (8, _D))
    return pl.pallas_call(
        _table_body,
        out_shape=jax.ShapeDtypeStruct((_VPAD, _D), jnp.float32),
    )(sym, W, b2)


@functools.cache
def _make_gather(tot, d):
    info = plsc.get_sparse_core_info()
    nc, ns = info.num_cores, info.num_subcores
    nw = nc * ns
    per_w = tot // nw
    ch = 1024
    ksub = ch // _IDX_PER_STREAM
    nch = per_w // ch
    rows_per_w = per_w // _IDX_PER_STREAM
    mesh = plsc.VectorSubcoreMesh(core_axis_name="c", subcore_axis_name="s")

    @functools.partial(
        pl.kernel,
        out_type=jax.ShapeDtypeStruct((tot, d), jnp.float32),
        mesh=mesh,
        scratch_types=[
            pltpu.VMEM((ksub, _IDX_PER_STREAM), jnp.int32),
            pltpu.VMEM((ch, d), jnp.float32),
            pltpu.SemaphoreType.DMA,
        ],
        compiler_params=pltpu.CompilerParams(use_tc_tiling_on_sc=False),
    )
    def gather(table_hbm, qr_hbm, out_hbm, idx_v, rows_v, sem):
        wid = lax.axis_index("s") * nc + lax.axis_index("c")
        base = wid * per_w
        row_base = wid * rows_per_w

        def chunk(g, carry):
            pltpu.sync_copy(qr_hbm.at[pl.ds(row_base + g * ksub, ksub)], idx_v)
            for j in range(ksub):
                pltpu.async_copy(
                    table_hbm.at[idx_v.at[j]],
                    rows_v.at[pl.ds(j * _IDX_PER_STREAM, _IDX_PER_STREAM)],
                    sem,
                ).wait()
            pltpu.sync_copy(rows_v, out_hbm.at[pl.ds(base + g * ch, ch)])
            return carry

        lax.fori_loop(0, nch, chunk, 0)

    return gather


def kernel(QR, symbols, W, b):
    bsz, seq = QR.shape
    tot = bsz * seq
    table = _fused_table(symbols, W, b)
    qr2 = QR.reshape(tot // _IDX_PER_STREAM, _IDX_PER_STREAM).astype(jnp.int32)
    out = _make_gather(tot, _D)(table, qr2)
    return out.reshape(bsz, seq, _D)
